# baseline (speedup 1.0000x reference)
import functools

import jax
import jax.numpy as jnp
from jax import lax
from jax.experimental import pallas as pl
from jax.experimental.pallas import tpu as pltpu
from jax.experimental.pallas import tpu_sc as plsc

N_USERS = 1000000
D = 32
B = 16384
NC = 2
NS = 16
NW = NC * NS
BPW = B // NW
GCHUNK = 128
NGC = BPW // GCHUNK
L = 16


def _perm(x, idx):
    dnums = lax.GatherDimensionNumbers(
        offset_dims=(), collapsed_slice_dims=(0,), start_index_map=(0,))
    return lax.gather(x, idx[:, None], dnums, slice_sizes=(1,),
                      mode=lax.GatherScatterMode.PROMISE_IN_BOUNDS)


@functools.partial(jax.jit, static_argnames=())
def _gmf_sc(users, items, embed_user, W):
    uidx = users.reshape(NW * NGC, GCHUNK).astype(jnp.int32)
    iidx = items.reshape(NW * NGC, GCHUNK).astype(jnp.int32)
    uidx = uidx.reshape(NW, NGC, GCHUNK)
    iidx = iidx.reshape(NW, NGC, GCHUNK)

    mesh = plsc.VectorSubcoreMesh(core_axis_name="c", subcore_axis_name="s")

    @functools.partial(
        pl.kernel,
        mesh=mesh,
        out_type=jax.ShapeDtypeStruct((B,), jnp.float32),
        compiler_params=pltpu.CompilerParams(use_tc_tiling_on_sc=False),
        scratch_types=[
            pltpu.VMEM((NGC, GCHUNK), jnp.int32),
            pltpu.VMEM((NGC, GCHUNK), jnp.int32),
            pltpu.VMEM((BPW, D), jnp.float32),
            pltpu.VMEM((BPW, D), jnp.float32),
            pltpu.VMEM((1, D), jnp.float32),
            pltpu.VMEM((BPW,), jnp.float32),
            pltpu.SemaphoreType.DMA,
        ],
    )
    def run(users_hbm, items_hbm, table_hbm, w_hbm, out_hbm,
            uidx_v, iidx_v, urows_v, irows_v, w_v, out_v, sem):
        wid = lax.axis_index("s") * NC + lax.axis_index("c")
        pltpu.sync_copy(users_hbm.at[wid], uidx_v)
        pltpu.sync_copy(items_hbm.at[wid], iidx_v)
        pltpu.sync_copy(w_hbm, w_v)

        copies = []
        for g in range(NGC):
            rows = pl.ds(g * GCHUNK, GCHUNK)
            copies.append(pltpu.async_copy(
                table_hbm.at[uidx_v.at[g]], urows_v.at[rows], sem))
            copies.append(pltpu.async_copy(
                table_hbm.at[iidx_v.at[g]], irows_v.at[rows], sem))
        for c in copies:
            c.wait()

        w0 = w_v[0, pl.ds(0, L)]
        w1 = w_v[0, pl.ds(L, L)]

        lanes = lax.iota(jnp.int32, L)
        rot8 = lanes ^ 8
        rot4 = lanes ^ 4
        rot2 = lanes ^ 2
        rot1 = lanes ^ 1

        def chunk_step(c, _):
            r0 = c * L
            acc = jnp.zeros((L,), jnp.float32)
            for l in range(L):
                r = r0 + l
                u0 = urows_v[r, pl.ds(0, L)]
                u1 = urows_v[r, pl.ds(L, L)]
                i0 = irows_v[r, pl.ds(0, L)]
                i1 = irows_v[r, pl.ds(L, L)]
                s = u0 * i0 * w0 + u1 * i1 * w1
                s = s + _perm(s, rot8)
                s = s + _perm(s, rot4)
                s = s + _perm(s, rot2)
                s = s + _perm(s, rot1)
                acc = jnp.where(lanes == l, s, acc)
            out_v[pl.ds(r0, L)] = acc
            return 0

        lax.fori_loop(0, BPW // L, chunk_step, 0)

        base = wid * BPW
        pltpu.sync_copy(out_v, out_hbm.at[pl.ds(base, BPW)])

    return run(uidx, iidx, embed_user, W)


def kernel(users, items, embed_user, W):
    return _gmf_sc(users, items, embed_user, W)
